# baseline (speedup 1.0000x reference)
import functools

import jax
import jax.numpy as jnp
from jax import lax
from jax.experimental import pallas as pl
from jax.experimental.pallas import tpu as pltpu
from jax.experimental.pallas import tpu_sc as plsc

_L = 16
_OUT_FLAT = 48
_X_FLAT = 12

_mesh = plsc.VectorSubcoreMesh(core_axis_name="c", subcore_axis_name="s")


@functools.partial(
    pl.kernel,
    mesh=_mesh,
    out_type=jax.ShapeDtypeStruct((_OUT_FLAT,), jnp.float32),
    scratch_types=[
        pltpu.VMEM((_L,), jnp.float32),
        pltpu.VMEM((_OUT_FLAT,), jnp.float32),
    ],
)
def _scatter_overwrite(x_hbm, out_hbm, x_v, buf_v):
    wid = lax.axis_index("s") * 2 + lax.axis_index("c")

    @pl.when(wid == 0)
    def _():
        pltpu.sync_copy(x_hbm, x_v)
        zero = jnp.zeros((_L,), jnp.float32)
        buf_v[pl.ds(0, _L)] = zero
        buf_v[pl.ds(_L, _L)] = zero
        buf_v[pl.ds(2 * _L, _L)] = zero
        vals = x_v[...]
        buf_v[pl.ds(_X_FLAT, _L)] = vals
        buf_v[pl.ds(2 * _X_FLAT, _L)] = vals
        pltpu.sync_copy(buf_v, out_hbm)


def kernel(x):
    x16 = jnp.zeros((_L,), jnp.float32).at[:_X_FLAT].set(x.reshape(-1))
    out = _scatter_overwrite(x16)
    return out.reshape(4, 2, 2, 3)


# speedup vs baseline: 1.0512x; 1.0512x over previous
import functools

import jax
import jax.numpy as jnp
from jax import lax
from jax.experimental import pallas as pl
from jax.experimental.pallas import tpu as pltpu
from jax.experimental.pallas import tpu_sc as plsc

_L = 16
_OUT_FLAT = 48
_X_FLAT = 12

_mesh = plsc.VectorSubcoreMesh(
    core_axis_name="c", subcore_axis_name="s", num_cores=1, num_subcores=1
)


@functools.partial(
    pl.kernel,
    mesh=_mesh,
    out_type=jax.ShapeDtypeStruct((_OUT_FLAT,), jnp.float32),
    scratch_types=[
        pltpu.VMEM((_L,), jnp.float32),
        pltpu.VMEM((_OUT_FLAT,), jnp.float32),
    ],
)
def _scatter_overwrite(x_hbm, out_hbm, x_v, buf_v):
    zero = jnp.zeros((_L,), jnp.float32)
    x_v[...] = zero
    pltpu.sync_copy(x_hbm, x_v.at[pl.ds(0, _X_FLAT)])
    buf_v[pl.ds(0, _L)] = zero
    buf_v[pl.ds(2 * _L, _L)] = zero
    vals = x_v[...]
    buf_v[pl.ds(_X_FLAT, _L)] = vals
    buf_v[pl.ds(2 * _X_FLAT, _L)] = vals
    pltpu.sync_copy(buf_v, out_hbm)


def kernel(x):
    out = _scatter_overwrite(x.reshape(-1))
    return out.reshape(4, 2, 2, 3)
